# Initial kernel scaffold; baseline (speedup 1.0000x reference)
import jax
import jax.numpy as jnp
from jax.experimental import pallas as pl


def kernel(features, xyz_0, xyz_1, xyz_2, xyz_3, neigh_idx_0, neigh_idx_1, neigh_idx_2, neigh_idx_3, sub_idx_0, sub_idx_1, sub_idx_2, sub_idx_3, interp_idx_0, interp_idx_1, interp_idx_2, interp_idx_3, params):
    raise NotImplementedError("write your pallas kernel here")


# speedup vs baseline: 2.1145x; 2.1145x over previous
import functools

import jax
import jax.numpy as jnp
from jax import lax
from jax.experimental import pallas as pl
from jax.experimental.pallas import tpu as pltpu
from jax.experimental.pallas import tpu_sc as plsc

NC, NS = 2, 16
NW = NC * NS
KNN = 16
NP = [50176, 12800, 3328, 1024, 256]
BN = [1024, 512, 256, 256, 256]
D_OUT = [16, 64, 128, 256]

@functools.cache
def _mesh():
    return plsc.VectorSubcoreMesh(
        core_axis_name="c", subcore_axis_name="s", num_cores=NC, num_subcores=NS)


def _lin_body(x_ref, w_ref, b_ref, o_ref, *, act):
    y = jnp.dot(x_ref[...], w_ref[...], preferred_element_type=jnp.float32, precision=lax.Precision.HIGHEST)
    y = y + b_ref[...]
    o_ref[...] = jnp.maximum(y, 0.0) if act else y


def _lin(x, w, b, act=True, bn=512):
    n, din = x.shape
    dout = w.shape[1]
    return pl.pallas_call(
        functools.partial(_lin_body, act=act),
        grid=(n // bn,),
        in_specs=[pl.BlockSpec((bn, din), lambda i: (i, 0)),
                  pl.BlockSpec((din, dout), lambda i: (0, 0)),
                  pl.BlockSpec((1, dout), lambda i: (0, 0))],
        out_specs=pl.BlockSpec((bn, dout), lambda i: (i, 0)),
        out_shape=jax.ShapeDtypeStruct((n, dout), jnp.float32),
    )(x, w, b.reshape(1, -1))


def _lin2_body(a_ref, b_ref, wa_ref, wb_ref, bias_ref, o_ref):
    y = jnp.dot(a_ref[...], wa_ref[...], preferred_element_type=jnp.float32, precision=lax.Precision.HIGHEST)
    y = y + jnp.dot(b_ref[...], wb_ref[...], preferred_element_type=jnp.float32, precision=lax.Precision.HIGHEST)
    o_ref[...] = jnp.maximum(y + bias_ref[...], 0.0)


def _lin2(a, b, wa, wb, bias, bn=512):
    n = a.shape[0]
    dout = wa.shape[1]
    return pl.pallas_call(
        _lin2_body,
        grid=(n // bn,),
        in_specs=[pl.BlockSpec((bn, a.shape[1]), lambda i: (i, 0)),
                  pl.BlockSpec((bn, b.shape[1]), lambda i: (i, 0)),
                  pl.BlockSpec((a.shape[1], dout), lambda i: (0, 0)),
                  pl.BlockSpec((b.shape[1], dout), lambda i: (0, 0)),
                  pl.BlockSpec((1, dout), lambda i: (0, 0))],
        out_specs=pl.BlockSpec((bn, dout), lambda i: (i, 0)),
        out_shape=jax.ShapeDtypeStruct((n, dout), jnp.float32),
    )(a, b, wa, wb, bias.reshape(1, -1))


def _fg0_body(x_ref, xyz_ref, w0_ref, b0_ref, w1_ref, b1_ref, wp_ref, o_ref):
    h = lax.dot_general(x_ref[...], w0_ref[...], (((0,), (0,)), ((), ())),
                        preferred_element_type=jnp.float32, precision=lax.Precision.HIGHEST)
    h = jnp.maximum(h + b0_ref[...], 0.0)
    f = jnp.dot(h, w1_ref[...], preferred_element_type=jnp.float32, precision=lax.Precision.HIGHEST)
    f = jnp.maximum(f + b1_ref[...], 0.0)
    g = jnp.dot(xyz_ref[...], wp_ref[...], preferred_element_type=jnp.float32, precision=lax.Precision.HIGHEST)
    o_ref[...] = jnp.concatenate([f, g], axis=1)


def _fg0(x6, xyz, w0, b0, w1, b1, wp, bn):
    n = xyz.shape[0]
    d = w1.shape[1]
    return pl.pallas_call(
        _fg0_body,
        grid=(n // bn,),
        in_specs=[pl.BlockSpec((6, bn), lambda i: (0, i)),
                  pl.BlockSpec((bn, 3), lambda i: (i, 0)),
                  pl.BlockSpec((6, 8), lambda i: (0, 0)),
                  pl.BlockSpec((1, 8), lambda i: (0, 0)),
                  pl.BlockSpec((8, d), lambda i: (0, 0)),
                  pl.BlockSpec((1, d), lambda i: (0, 0)),
                  pl.BlockSpec((3, d), lambda i: (0, 0))],
        out_specs=pl.BlockSpec((bn, 2 * d), lambda i: (i, 0)),
        out_shape=jax.ShapeDtypeStruct((n, 2 * d), jnp.float32),
    )(x6, xyz, w0, b0.reshape(1, -1), w1, b1.reshape(1, -1), wp)


def _fg_body(f_ref, xyz_ref, w1_ref, b1_ref, wp_ref, o_ref):
    f = jnp.dot(f_ref[...], w1_ref[...], preferred_element_type=jnp.float32, precision=lax.Precision.HIGHEST)
    f = jnp.maximum(f + b1_ref[...], 0.0)
    g = jnp.dot(xyz_ref[...], wp_ref[...], preferred_element_type=jnp.float32, precision=lax.Precision.HIGHEST)
    o_ref[...] = jnp.concatenate([f, g], axis=1)


def _fg(fin, xyz, w1, b1, wp, bn):
    n, din = fin.shape
    d = w1.shape[1]
    return pl.pallas_call(
        _fg_body,
        grid=(n // bn,),
        in_specs=[pl.BlockSpec((bn, din), lambda i: (i, 0)),
                  pl.BlockSpec((bn, 3), lambda i: (i, 0)),
                  pl.BlockSpec((din, d), lambda i: (0, 0)),
                  pl.BlockSpec((1, d), lambda i: (0, 0)),
                  pl.BlockSpec((3, d), lambda i: (0, 0))],
        out_specs=pl.BlockSpec((bn, 2 * d), lambda i: (i, 0)),
        out_shape=jax.ShapeDtypeStruct((n, 2 * d), jnp.float32),
    )(fin, xyz, w1, b1.reshape(1, -1), wp)


def _head_body(x_ref, w1_ref, b1_ref, w2_ref, b2_ref, w3_ref, b3_ref, o_ref):
    f1 = jnp.dot(x_ref[...], w1_ref[...], preferred_element_type=jnp.float32, precision=lax.Precision.HIGHEST)
    f1 = jnp.maximum(f1 + b1_ref[...], 0.0)
    f2 = jnp.dot(f1, w2_ref[...], preferred_element_type=jnp.float32, precision=lax.Precision.HIGHEST)
    f2 = jnp.maximum(f2 + b2_ref[...], 0.0)
    ot = lax.dot_general(w3_ref[...], f2, (((0,), (1,)), ((), ())),
                         preferred_element_type=jnp.float32, precision=lax.Precision.HIGHEST)
    o_ref[...] = ot + b3_ref[...]


def _head(x, w1, b1, w2, b2, w3, b3, bn):
    n = x.shape[0]
    return pl.pallas_call(
        _head_body,
        grid=(n // bn,),
        in_specs=[pl.BlockSpec((bn, x.shape[1]), lambda i: (i, 0)),
                  pl.BlockSpec(w1.shape, lambda i: (0, 0)),
                  pl.BlockSpec((1, w1.shape[1]), lambda i: (0, 0)),
                  pl.BlockSpec(w2.shape, lambda i: (0, 0)),
                  pl.BlockSpec((1, w2.shape[1]), lambda i: (0, 0)),
                  pl.BlockSpec(w3.shape, lambda i: (0, 0)),
                  pl.BlockSpec((13, 1), lambda i: (0, 0))],
        out_specs=pl.BlockSpec((13, bn), lambda i: (0, i)),
        out_shape=jax.ShapeDtypeStruct((13, n), jnp.float32),
    )(x, w1, b1.reshape(1, -1), w2, b2.reshape(1, -1), w3, b3.reshape(-1, 1))


def _make_agg(n_src, n_out, d, d_table):
    ppw = n_out // NW
    c = 8
    nchunks = ppw // c

    def body(table_ref, idx_ref, bp_ref, out_ref,
             idx_v, rows_v, cen_v, bp_v, out_v, sem):
        wid = lax.axis_index("s") * NC + lax.axis_index("c")
        base = wid * ppw
        pltpu.sync_copy(bp_ref, bp_v)

        def chunk(ci, carry):
            pbase = base + ci * c
            pltpu.sync_copy(idx_ref.at[pl.ds(pl.multiple_of(pbase * KNN, 128),
                                             c * KNN)], idx_v)
            pltpu.async_copy(table_ref.at[idx_v], rows_v, sem).wait()
            pltpu.sync_copy(table_ref.at[pl.ds(pl.multiple_of(pbase, 8), c)],
                            cen_v)

            def pbody(nn, carry2):
                for j in range(d // 16):
                    slf = pl.ds(j * 16, 16)
                    slg = pl.ds(d + j * 16, 16)
                    gc = cen_v[nn, slg]
                    bpv = bp_v[slf]

                    def kbody(kk, acc):
                        rf = rows_v[nn * KNN + kk, slf]
                        rg = rows_v[nn * KNN + kk, slg]
                        return jnp.maximum(
                            acc, rf + jnp.maximum(rg - gc + bpv, 0.0))

                    acc0 = rows_v[nn * KNN, slf] + jnp.maximum(
                        rows_v[nn * KNN, slg] - gc + bpv, 0.0)
                    acc = lax.fori_loop(1, KNN, kbody, acc0, unroll=5)
                    out_v[nn, slf] = acc
                return carry2

            lax.fori_loop(0, c, pbody, 0)
            pltpu.sync_copy(out_v, out_ref.at[pl.ds(pl.multiple_of(pbase, 8), c)])
            return carry

        lax.fori_loop(0, nchunks, chunk, 0)

    return pl.kernel(
        body,
        out_type=jax.ShapeDtypeStruct((n_out, d), jnp.float32),
        mesh=_mesh(),
        compiler_params=pltpu.CompilerParams(use_tc_tiling_on_sc=False),
        scratch_types=[
            pltpu.VMEM((c * KNN,), jnp.int32),
            pltpu.VMEM((c * KNN, d_table), jnp.float32),
            pltpu.VMEM((c, d_table), jnp.float32),
            pltpu.VMEM((d,), jnp.float32),
            pltpu.VMEM((c, d), jnp.float32),
            pltpu.SemaphoreType.DMA,
        ])


def _make_pool(n_src, n_out, d):
    ppw = n_out // NW
    c = 8
    nchunks = ppw // c

    def body(table_ref, idx_ref, out_ref, idx_v, rows_v, out_v, sem):
        wid = lax.axis_index("s") * NC + lax.axis_index("c")
        base = wid * ppw

        def chunk(ci, carry):
            pbase = base + ci * c
            pltpu.sync_copy(idx_ref.at[pl.ds(pl.multiple_of(pbase * KNN, 128),
                                             c * KNN)], idx_v)
            pltpu.async_copy(table_ref.at[idx_v], rows_v, sem).wait()

            def pbody(nn, carry2):
                for j in range(d // 16):
                    slf = pl.ds(j * 16, 16)

                    def kbody(kk, acc):
                        return jnp.maximum(acc, rows_v[nn * KNN + kk, slf])

                    acc = lax.fori_loop(1, KNN, kbody, rows_v[nn * KNN, slf],
                                        unroll=5)
                    out_v[nn, slf] = acc
                return carry2

            lax.fori_loop(0, c, pbody, 0)
            pltpu.sync_copy(out_v, out_ref.at[pl.ds(pl.multiple_of(pbase, 8), c)])
            return carry

        lax.fori_loop(0, nchunks, chunk, 0)

    return pl.kernel(
        body,
        out_type=jax.ShapeDtypeStruct((n_out, d), jnp.float32),
        mesh=_mesh(),
        compiler_params=pltpu.CompilerParams(use_tc_tiling_on_sc=False),
        scratch_types=[
            pltpu.VMEM((c * KNN,), jnp.int32),
            pltpu.VMEM((c * KNN, d), jnp.float32),
            pltpu.VMEM((c, d), jnp.float32),
            pltpu.SemaphoreType.DMA,
        ])


def _make_interp(n_src, n_out, d, cs):
    ppw = n_out // NW
    nchunks = ppw // cs

    def body(table_ref, idx_ref, out_ref, idx_v, rows_v, sem):
        wid = lax.axis_index("s") * NC + lax.axis_index("c")
        base = wid * ppw

        def chunk(ci, carry):
            pbase = base + ci * cs
            pltpu.sync_copy(idx_ref.at[pl.ds(pl.multiple_of(pbase, 8), cs)],
                            idx_v)
            pltpu.async_copy(table_ref.at[idx_v], rows_v, sem).wait()
            pltpu.sync_copy(rows_v, out_ref.at[pl.ds(pl.multiple_of(pbase, 8), cs)])
            return carry

        lax.fori_loop(0, nchunks, chunk, 0)

    return pl.kernel(
        body,
        out_type=jax.ShapeDtypeStruct((n_out, d), jnp.float32),
        mesh=_mesh(),
        compiler_params=pltpu.CompilerParams(use_tc_tiling_on_sc=False),
        scratch_types=[
            pltpu.VMEM((cs,), jnp.int32),
            pltpu.VMEM((cs, d), jnp.float32),
            pltpu.SemaphoreType.DMA,
        ])


def _pad_rows(x, n):
    return jnp.pad(x, ((0, n - x.shape[0]),) + ((0, 0),) * (x.ndim - 1))


def kernel(features, xyz_0, xyz_1, xyz_2, xyz_3,
           neigh_idx_0, neigh_idx_1, neigh_idx_2, neigh_idx_3,
           sub_idx_0, sub_idx_1, sub_idx_2, sub_idx_3,
           interp_idx_0, interp_idx_1, interp_idx_2, interp_idx_3, params):
    p = params
    xyzs = [xyz_0, xyz_1, xyz_2, xyz_3]
    neighs = [neigh_idx_0, neigh_idx_1, neigh_idx_2, neigh_idx_3]
    subs = [sub_idx_0, sub_idx_1, sub_idx_2, sub_idx_3]
    interps = [interp_idx_0, interp_idx_1, interp_idx_2, interp_idx_3]

    xyzp = [_pad_rows(x[0], NP[i]) for i, x in enumerate(xyzs)]
    neighp = [_pad_rows(x[0].astype(jnp.int32), NP[i]).reshape(-1)
              for i, x in enumerate(neighs)]
    subp = [_pad_rows(x[0].astype(jnp.int32), NP[i + 1]).reshape(-1)
            for i, x in enumerate(subs)]
    interpp = [_pad_rows(x[0, :, 0].astype(jnp.int32), NP[i])
               for i, x in enumerate(interps)]
    feats = jnp.pad(features[0], ((0, 0), (0, NP[0] - features.shape[2])))

    enc = []
    f = None
    fg = None
    for i in range(4):
        lp = p['lfa'][i]
        d = D_OUT[i]
        if i == 0:
            fg = _fg0(feats, xyzp[0], p['fc0_W'], p['fc0_b'],
                      lp['W1'], lp['b1'], lp['Wp'], BN[0])
        else:
            fg = _fg(f, xyzp[i], lp['W1'], lp['b1'], lp['Wp'], BN[i])
        agg = _make_agg(NP[i], NP[i], d, 2 * d)(fg, neighp[i], lp['bp'])
        fe = _lin(agg, lp['W2'], lp['b2'], act=True, bn=BN[i])
        fs = _make_pool(NP[i], NP[i + 1], 2 * d)(fe, subp[i])
        if i == 0:
            enc.append(fe)
        enc.append(fs)
        f = fs

    f = _lin(f, p['dec0_W'], p['dec0_b'], act=True, bn=BN[4])
    interp_cs = [112, 80, 104, 32]
    for j in range(4):
        lvl = 3 - j
        fi = _make_interp(NP[lvl + 1], NP[lvl], f.shape[1], interp_cs[lvl])(
            f, interpp[lvl])
        skip = enc[-j - 2]
        da = skip.shape[1]
        w = p['dec_W'][j]
        f = _lin2(skip, fi, w[:da], w[da:], p['dec_b'][j], bn=BN[lvl])

    logits_t = _head(f, p['fc1_W'], p['fc1_b'], p['fc2_W'], p['fc2_b'],
                     p['fc3_W'], p['fc3_b'], BN[0])
    return logits_t[:, :features.shape[2]][None]
